# baseline (speedup 1.0000x reference)
import functools

import jax
import jax.numpy as jnp
from jax import lax
from jax.experimental import pallas as pl
from jax.experimental.pallas import tpu as pltpu
from jax.experimental.pallas import tpu_sc as plsc

_VOCAB = 1000000
_D = 32
_BATCH = 16384
_HIST = 50
_NC = 2
_NS = 16
_NW = _NC * _NS
_BBLK = _BATCH // 128
_CPW = _BBLK // _NW
_NU = _HIST * _CPW

_mesh = plsc.VectorSubcoreMesh(core_axis_name="c", subcore_axis_name="s")


@functools.partial(
    pl.kernel,
    out_type=jax.ShapeDtypeStruct((_HIST, _D // 8, 128, 8, 128), jnp.float32),
    mesh=_mesh,
    scratch_types=[
        pltpu.VMEM((_HIST, _CPW * 128), jnp.int32),
        pltpu.VMEM((2, 128, _D), jnp.float32),
        pltpu.VMEM((2, _D // 8, 8, 128), jnp.float32),
        pltpu.SemaphoreType.DMA,
        pltpu.SemaphoreType.DMA,
        pltpu.SemaphoreType.DMA,
        pltpu.SemaphoreType.DMA,
    ],
    compiler_params=pltpu.CompilerParams(use_tc_tiling_on_sc=False, needs_layout_passes=False),
)
def _gather(ids_hbm, table_hbm, out_hbm, idx_v, rows_v, tblk_v,
            gsem0, gsem1, osem0, osem1):
    wid = lax.axis_index("s") * _NC + lax.axis_index("c")
    col0 = wid * (_CPW * 128)
    pltpu.sync_copy(ids_hbm.at[:, pl.ds(col0, _CPW * 128)], idx_v)

    lanes = lax.iota(jnp.int32, 16)
    rvecs = [lanes + (bg * 16) for bg in range(8)]

    def fire_gather(u, half, sem):
        t = u // _CPW
        c = u % _CPW
        pltpu.async_copy(table_hbm.at[idx_v.at[t, pl.ds(c * 128, 128)]],
                         rows_v.at[half], sem)

    def drain_gather(u, half, sem):
        t = u // _CPW
        c = u % _CPW
        pltpu.make_async_copy(table_hbm.at[idx_v.at[t, pl.ds(c * 128, 128)]],
                              rows_v.at[half], sem).wait()

    def transpose(half):
        for jb in range(_D // 8):
            for ji in range(8):
                cvec = jnp.full((16,), jb * 8 + ji, jnp.int32)
                for bg in range(8):
                    vals = plsc.load_gather(rows_v.at[half], [rvecs[bg], cvec])
                    tblk_v.at[half, jb, ji][pl.ds(bg * 16, 16)] = vals

    def fire_out(u, half, sem):
        t = u // _CPW
        c = u % _CPW
        pltpu.async_copy(tblk_v.at[half],
                         out_hbm.at[t, :, wid * _CPW + c], sem)

    def drain_out(u, half, sem):
        t = u // _CPW
        c = u % _CPW
        pltpu.make_async_copy(tblk_v.at[half],
                              out_hbm.at[t, :, wid * _CPW + c], sem).wait()

    fire_gather(0, 0, gsem0)

    @pl.loop(0, _NU // 2)
    def _pair(up):
        u0 = 2 * up
        u1 = u0 + 1
        drain_gather(u0, 0, gsem0)
        fire_gather(u1, 1, gsem1)
        transpose(0)
        fire_out(u0, 0, osem0)
        drain_gather(u1, 1, gsem1)

        @pl.when(up + 1 < _NU // 2)
        def _():
            fire_gather(u0 + 2, 0, gsem0)

        transpose(1)
        fire_out(u1, 1, osem1)
        drain_out(u0, 0, osem0)
        drain_out(u1, 1, osem1)


def kernel(ids, length, table):
    del length
    out5 = _gather(ids.T, table)
    return jnp.transpose(out5, (2, 4, 0, 1, 3)).reshape(_BATCH, _HIST, _D)


# speedup vs baseline: 1.0685x; 1.0685x over previous
import functools

import jax
import jax.numpy as jnp
from jax import lax
from jax.experimental import pallas as pl
from jax.experimental.pallas import tpu as pltpu
from jax.experimental.pallas import tpu_sc as plsc

_VOCAB = 1000000
_D = 32
_BATCH = 16384
_HIST = 50
_NC = 2
_NS = 16
_NW = _NC * _NS
_BBLK = _BATCH // 128
_CPW = _BBLK // _NW
_NU = _HIST * _CPW

_mesh = plsc.VectorSubcoreMesh(core_axis_name="c", subcore_axis_name="s")


@functools.partial(
    pl.kernel,
    out_type=jax.ShapeDtypeStruct((_HIST, _D // 8, 128, 8, 128), jnp.float32),
    mesh=_mesh,
    scratch_types=[
        pltpu.VMEM((_HIST, _CPW * 128), jnp.int32),
        pltpu.VMEM((2, 128, _D), jnp.float32),
        pltpu.VMEM((2, _D, 128), jnp.float32),
        pltpu.SemaphoreType.DMA,
        pltpu.SemaphoreType.DMA,
        pltpu.SemaphoreType.DMA,
        pltpu.SemaphoreType.DMA,
    ],
    compiler_params=pltpu.CompilerParams(use_tc_tiling_on_sc=False, needs_layout_passes=False),
)
def _gather(ids_hbm, table_hbm, out_hbm, idx_v, rows_v, tblk_v,
            gsem0, gsem1, osem0, osem1):
    wid = lax.axis_index("s") * _NC + lax.axis_index("c")
    col0 = wid * (_CPW * 128)
    pltpu.sync_copy(ids_hbm.at[:, pl.ds(col0, _CPW * 128)], idx_v)

    lanes = lax.iota(jnp.int32, 16)
    rvecs = [lanes + (bg * 16) for bg in range(8)]

    def fire_gather(u, half, sem):
        t = u // _CPW
        c = u % _CPW
        pltpu.async_copy(table_hbm.at[idx_v.at[t, pl.ds(c * 128, 128)]],
                         rows_v.at[half], sem)

    def drain_gather(u, half, sem):
        t = u // _CPW
        c = u % _CPW
        pltpu.make_async_copy(table_hbm.at[idx_v.at[t, pl.ds(c * 128, 128)]],
                              rows_v.at[half], sem).wait()

    def transpose(half):
        @pl.loop(0, _D)
        def _j(j):
            cvec = jnp.zeros((16,), jnp.int32) + j
            for bg in range(8):
                vals = plsc.load_gather(rows_v.at[half], [rvecs[bg], cvec])
                tblk_v.at[half, j][pl.ds(bg * 16, 16)] = vals

    def fire_out(u, half, sem):
        t = u // _CPW
        c = u % _CPW
        for jb in range(_D // 8):
            pltpu.async_copy(tblk_v.at[half, pl.ds(jb * 8, 8)],
                             out_hbm.at[t, jb, wid * _CPW + c], sem)

    def drain_out(u, half, sem):
        t = u // _CPW
        c = u % _CPW
        for jb in range(_D // 8):
            pltpu.make_async_copy(tblk_v.at[half, pl.ds(jb * 8, 8)],
                                  out_hbm.at[t, jb, wid * _CPW + c], sem).wait()

    fire_gather(0, 0, gsem0)

    @pl.loop(0, _NU // 2)
    def _pair(up):
        u0 = 2 * up
        u1 = u0 + 1
        drain_gather(u0, 0, gsem0)
        fire_gather(u1, 1, gsem1)
        transpose(0)
        fire_out(u0, 0, osem0)
        drain_gather(u1, 1, gsem1)

        @pl.when(up + 1 < _NU // 2)
        def _():
            fire_gather(u0 + 2, 0, gsem0)

        transpose(1)
        fire_out(u1, 1, osem1)
        drain_out(u0, 0, osem0)
        drain_out(u1, 1, osem1)


def kernel(ids, length, table):
    del length
    out5 = _gather(ids.T, table)
    return jnp.transpose(out5, (2, 4, 0, 1, 3)).reshape(_BATCH, _HIST, _D)


# speedup vs baseline: 1.3206x; 1.2359x over previous
import functools

import jax
import jax.numpy as jnp
from jax import lax
from jax.experimental import pallas as pl
from jax.experimental.pallas import tpu as pltpu
from jax.experimental.pallas import tpu_sc as plsc

_VOCAB = 1000000
_D = 32
_BATCH = 16384
_HIST = 50
_NC = 2
_NS = 16
_NW = _NC * _NS
_BBLK = _BATCH // 128
_CPW = _BBLK // _NW
_NU = _HIST * _CPW

_mesh = plsc.VectorSubcoreMesh(core_axis_name="c", subcore_axis_name="s")


@functools.partial(
    pl.kernel,
    out_type=jax.ShapeDtypeStruct((_HIST, _D // 8, 128, 8, 128), jnp.float32),
    mesh=_mesh,
    scratch_types=[
        pltpu.VMEM((_HIST, _CPW * 128), jnp.int32),
        pltpu.VMEM((2, 128, _D), jnp.float32),
        pltpu.VMEM((2, _D, 128), jnp.float32),
        pltpu.SemaphoreType.DMA,
        pltpu.SemaphoreType.DMA,
        pltpu.SemaphoreType.DMA,
        pltpu.SemaphoreType.DMA,
    ],
    compiler_params=pltpu.CompilerParams(use_tc_tiling_on_sc=False, needs_layout_passes=False),
)
def _gather(ids_hbm, table_hbm, out_hbm, idx_v, rows_v, tblk_v,
            gsem0, gsem1, osem0, osem1):
    wid = lax.axis_index("s") * _NC + lax.axis_index("c")
    col0 = wid * (_CPW * 128)
    pltpu.sync_copy(ids_hbm.at[:, pl.ds(col0, _CPW * 128)], idx_v)

    lanes = lax.iota(jnp.int32, 16)
    rvecs = [lanes + (bg * 16) for bg in range(8)]

    def fire_gather(u, half, sem):
        t = u // _CPW
        c = u % _CPW
        pltpu.async_copy(table_hbm.at[idx_v.at[t, pl.ds(c * 128, 128)]],
                         rows_v.at[half], sem)

    def drain_gather(u, half, sem):
        t = u // _CPW
        c = u % _CPW
        pltpu.make_async_copy(table_hbm.at[idx_v.at[t, pl.ds(c * 128, 128)]],
                              rows_v.at[half], sem).wait()

    def transpose(half):
        @pl.loop(0, _D // 4)
        def _j(jq):
            j0 = 4 * jq
            cvecs = [jnp.zeros((16,), jnp.int32) + (j0 + q) for q in range(4)]
            for bg in range(8):
                vals = [plsc.load_gather(rows_v.at[half], [rvecs[bg], cvecs[q]])
                        for q in range(4)]
                for q in range(4):
                    tblk_v.at[half, j0 + q][pl.ds(bg * 16, 16)] = vals[q]

    def fire_out(u, half, sem):
        t = u // _CPW
        c = u % _CPW
        for jb in range(_D // 8):
            pltpu.async_copy(tblk_v.at[half, pl.ds(jb * 8, 8)],
                             out_hbm.at[t, jb, wid * _CPW + c], sem)

    def drain_out(u, half, sem):
        t = u // _CPW
        c = u % _CPW
        for jb in range(_D // 8):
            pltpu.make_async_copy(tblk_v.at[half, pl.ds(jb * 8, 8)],
                                  out_hbm.at[t, jb, wid * _CPW + c], sem).wait()

    fire_gather(0, 0, gsem0)

    @pl.loop(0, _NU // 2)
    def _pair(up):
        u0 = 2 * up
        u1 = u0 + 1
        drain_gather(u0, 0, gsem0)
        fire_gather(u1, 1, gsem1)
        transpose(0)
        fire_out(u0, 0, osem0)
        drain_gather(u1, 1, gsem1)

        @pl.when(up + 1 < _NU // 2)
        def _():
            fire_gather(u0 + 2, 0, gsem0)

        transpose(1)
        fire_out(u1, 1, osem1)
        drain_out(u0, 0, osem0)
        drain_out(u1, 1, osem1)


def kernel(ids, length, table):
    del length
    out5 = _gather(ids.T, table)
    return jnp.transpose(out5, (2, 4, 0, 1, 3)).reshape(_BATCH, _HIST, _D)


# speedup vs baseline: 1.3570x; 1.0276x over previous
import functools

import jax
import jax.numpy as jnp
from jax import lax
from jax.experimental import pallas as pl
from jax.experimental.pallas import tpu as pltpu
from jax.experimental.pallas import tpu_sc as plsc

_VOCAB = 1000000
_D = 32
_BATCH = 16384
_HIST = 50
_NC = 2
_NS = 16
_NW = _NC * _NS
_BBLK = _BATCH // 128
_CPW = _BBLK // _NW
_NU = _HIST * _CPW

_mesh = plsc.VectorSubcoreMesh(core_axis_name="c", subcore_axis_name="s")


@functools.partial(
    pl.kernel,
    out_type=jax.ShapeDtypeStruct((_HIST, _D // 8, 128, 8, 128), jnp.float32),
    mesh=_mesh,
    scratch_types=[
        pltpu.VMEM((_HIST, _CPW * 128), jnp.int32),
        pltpu.VMEM((2, 128, _D), jnp.float32),
        pltpu.VMEM((2, _D, 128), jnp.float32),
        pltpu.SemaphoreType.DMA,
        pltpu.SemaphoreType.DMA,
        pltpu.SemaphoreType.DMA,
        pltpu.SemaphoreType.DMA,
    ],
    compiler_params=pltpu.CompilerParams(use_tc_tiling_on_sc=False, needs_layout_passes=False),
)
def _gather(ids_hbm, table_hbm, out_hbm, idx_v, rows_v, tblk_v,
            gsem0, gsem1, osem0, osem1):
    wid = lax.axis_index("s") * _NC + lax.axis_index("c")
    col0 = wid * (_CPW * 128)
    pltpu.sync_copy(ids_hbm.at[:, pl.ds(col0, _CPW * 128)], idx_v)

    lanes = lax.iota(jnp.int32, 16)
    rvecs = [lanes + (bg * 16) for bg in range(8)]

    def fire_gather(u, half, sem):
        t = u // _CPW
        c = u % _CPW
        pltpu.async_copy(table_hbm.at[idx_v.at[t, pl.ds(c * 128, 128)]],
                         rows_v.at[half], sem)

    def drain_gather(u, half, sem):
        t = u // _CPW
        c = u % _CPW
        pltpu.make_async_copy(table_hbm.at[idx_v.at[t, pl.ds(c * 128, 128)]],
                              rows_v.at[half], sem).wait()

    def transpose(half):
        @pl.loop(0, _D // 8)
        def _j(jq):
            j0 = 8 * jq
            cvecs = [jnp.zeros((16,), jnp.int32) + (j0 + q) for q in range(8)]
            for bg in range(8):
                vals = [plsc.load_gather(rows_v.at[half], [rvecs[bg], cvecs[q]])
                        for q in range(8)]
                for q in range(8):
                    tblk_v.at[half, j0 + q][pl.ds(bg * 16, 16)] = vals[q]

    def fire_out(u, half, sem):
        t = u // _CPW
        c = u % _CPW
        for jb in range(_D // 8):
            pltpu.async_copy(tblk_v.at[half, pl.ds(jb * 8, 8)],
                             out_hbm.at[t, jb, wid * _CPW + c], sem)

    def drain_out(u, half, sem):
        t = u // _CPW
        c = u % _CPW
        for jb in range(_D // 8):
            pltpu.make_async_copy(tblk_v.at[half, pl.ds(jb * 8, 8)],
                                  out_hbm.at[t, jb, wid * _CPW + c], sem).wait()

    fire_gather(0, 0, gsem0)

    @pl.loop(0, _NU // 2)
    def _pair(up):
        u0 = 2 * up
        u1 = u0 + 1
        drain_gather(u0, 0, gsem0)
        fire_gather(u1, 1, gsem1)
        transpose(0)
        fire_out(u0, 0, osem0)
        drain_gather(u1, 1, gsem1)

        @pl.when(up + 1 < _NU // 2)
        def _():
            fire_gather(u0 + 2, 0, gsem0)

        transpose(1)
        fire_out(u1, 1, osem1)
        drain_out(u0, 0, osem0)
        drain_out(u1, 1, osem1)


def kernel(ids, length, table):
    del length
    out5 = _gather(ids.T, table)
    return jnp.transpose(out5, (2, 4, 0, 1, 3)).reshape(_BATCH, _HIST, _D)


# speedup vs baseline: 1.3609x; 1.0029x over previous
import functools

import jax
import jax.numpy as jnp
from jax import lax
from jax.experimental import pallas as pl
from jax.experimental.pallas import tpu as pltpu
from jax.experimental.pallas import tpu_sc as plsc

_VOCAB = 1000000
_D = 32
_BATCH = 16384
_HIST = 50
_NC = 2
_NS = 16
_NW = _NC * _NS
_BBLK = _BATCH // 128
_CPW = _BBLK // _NW
_NU = _HIST * _CPW

_mesh = plsc.VectorSubcoreMesh(core_axis_name="c", subcore_axis_name="s")


@functools.partial(
    pl.kernel,
    out_type=jax.ShapeDtypeStruct((_HIST, _D // 8, 128, 8, 128), jnp.float32),
    mesh=_mesh,
    scratch_types=[
        pltpu.VMEM((_HIST, _CPW * 128), jnp.int32),
        pltpu.VMEM((2, 128, _D), jnp.float32),
        pltpu.VMEM((2, _D // 8, 8, 128), jnp.float32),
        pltpu.SemaphoreType.DMA,
        pltpu.SemaphoreType.DMA,
        pltpu.SemaphoreType.DMA,
        pltpu.SemaphoreType.DMA,
    ],
    compiler_params=pltpu.CompilerParams(use_tc_tiling_on_sc=False, needs_layout_passes=False),
)
def _gather(ids_hbm, table_hbm, out_hbm, idx_v, rows_v, tblk_v,
            gsem0, gsem1, osem0, osem1):
    wid = lax.axis_index("s") * _NC + lax.axis_index("c")
    col0 = wid * (_CPW * 128)
    pltpu.sync_copy(ids_hbm.at[:, pl.ds(col0, _CPW * 128)], idx_v)

    lanes = lax.iota(jnp.int32, 16)
    rvecs = [lanes + (bg * 16) for bg in range(8)]

    def fire_gather(u, half, sem):
        t = u // _CPW
        c = u % _CPW
        pltpu.async_copy(table_hbm.at[idx_v.at[t, pl.ds(c * 128, 128)]],
                         rows_v.at[half], sem)

    def drain_gather(u, half, sem):
        t = u // _CPW
        c = u % _CPW
        pltpu.make_async_copy(table_hbm.at[idx_v.at[t, pl.ds(c * 128, 128)]],
                              rows_v.at[half], sem).wait()

    def transpose(half):
        @pl.loop(0, _D // 8)
        def _j(jq):
            j0 = 8 * jq
            cvecs = [jnp.zeros((16,), jnp.int32) + (j0 + q) for q in range(8)]
            for bg in range(8):
                vals = [plsc.load_gather(rows_v.at[half], [rvecs[bg], cvecs[q]])
                        for q in range(8)]
                for q in range(8):
                    tblk_v.at[half, jq, q][pl.ds(bg * 16, 16)] = vals[q]

    def fire_out(u, half, sem):
        t = u // _CPW
        c = u % _CPW
        pltpu.async_copy(tblk_v.at[half],
                         out_hbm.at[t, :, wid * _CPW + c], sem)

    def drain_out(u, half, sem):
        t = u // _CPW
        c = u % _CPW
        pltpu.make_async_copy(tblk_v.at[half],
                              out_hbm.at[t, :, wid * _CPW + c], sem).wait()

    fire_gather(0, 0, gsem0)

    @pl.loop(0, _NU // 2)
    def _pair(up):
        u0 = 2 * up
        u1 = u0 + 1
        drain_gather(u0, 0, gsem0)
        fire_gather(u1, 1, gsem1)
        transpose(0)
        fire_out(u0, 0, osem0)
        drain_gather(u1, 1, gsem1)

        @pl.when(up + 1 < _NU // 2)
        def _():
            fire_gather(u0 + 2, 0, gsem0)

        transpose(1)
        fire_out(u1, 1, osem1)
        drain_out(u0, 0, osem0)
        drain_out(u1, 1, osem1)


def kernel(ids, length, table):
    del length
    out5 = _gather(ids.T, table)
    return jnp.transpose(out5, (2, 4, 0, 1, 3)).reshape(_BATCH, _HIST, _D)


# speedup vs baseline: 1.3843x; 1.0172x over previous
import functools

import jax
import jax.numpy as jnp
from jax import lax
from jax.experimental import pallas as pl
from jax.experimental.pallas import tpu as pltpu
from jax.experimental.pallas import tpu_sc as plsc

_VOCAB = 1000000
_D = 32
_BATCH = 16384
_HIST = 50
_NC = 2
_NS = 16
_NW = _NC * _NS
_BBLK = _BATCH // 128
_CPW = _BBLK // _NW
_NU = _HIST * _CPW

_mesh = plsc.VectorSubcoreMesh(core_axis_name="c", subcore_axis_name="s")


@functools.partial(
    pl.kernel,
    out_type=jax.ShapeDtypeStruct((_HIST, _D // 8, 128, 8, 128), jnp.float32),
    mesh=_mesh,
    scratch_types=[
        pltpu.VMEM((_HIST, _CPW * 128), jnp.int32),
        pltpu.VMEM((2, 128, _D), jnp.float32),
        pltpu.VMEM((2, _D // 8, 8, 128), jnp.float32),
        pltpu.SemaphoreType.DMA,
        pltpu.SemaphoreType.DMA,
        pltpu.SemaphoreType.DMA,
        pltpu.SemaphoreType.DMA,
    ],
    compiler_params=pltpu.CompilerParams(use_tc_tiling_on_sc=False, needs_layout_passes=False),
)
def _gather(ids_hbm, table_hbm, out_hbm, idx_v, rows_v, tblk_v,
            gsem0, gsem1, osem0, osem1):
    wid = lax.axis_index("s") * _NC + lax.axis_index("c")
    col0 = wid * (_CPW * 128)
    pltpu.sync_copy(ids_hbm.at[:, pl.ds(col0, _CPW * 128)], idx_v)

    lanes = lax.iota(jnp.int32, 16)
    rvecs = [lanes + (bg * 16) for bg in range(8)]

    def fire_gather(u, half, sem):
        t = u // _CPW
        c = u % _CPW
        pltpu.async_copy(table_hbm.at[idx_v.at[t, pl.ds(c * 128, 128)]],
                         rows_v.at[half], sem)

    def drain_gather(u, half, sem):
        t = u // _CPW
        c = u % _CPW
        pltpu.make_async_copy(table_hbm.at[idx_v.at[t, pl.ds(c * 128, 128)]],
                              rows_v.at[half], sem).wait()

    def transpose(half):
        @pl.loop(0, _D // 16)
        def _j(jq):
            j0 = 16 * jq
            cvecs = [jnp.zeros((16,), jnp.int32) + (j0 + q) for q in range(16)]
            for bg in range(8):
                vals = [plsc.load_gather(rows_v.at[half], [rvecs[bg], cvecs[q]])
                        for q in range(16)]
                for q in range(16):
                    tblk_v.at[half, 2 * jq + q // 8, q % 8][pl.ds(bg * 16, 16)] = vals[q]

    def fire_out(u, half, sem):
        t = u // _CPW
        c = u % _CPW
        pltpu.async_copy(tblk_v.at[half],
                         out_hbm.at[t, :, wid * _CPW + c], sem)

    def drain_out(u, half, sem):
        t = u // _CPW
        c = u % _CPW
        pltpu.make_async_copy(tblk_v.at[half],
                              out_hbm.at[t, :, wid * _CPW + c], sem).wait()

    fire_gather(0, 0, gsem0)

    @pl.loop(0, _NU // 2)
    def _pair(up):
        u0 = 2 * up
        u1 = u0 + 1
        drain_gather(u0, 0, gsem0)
        fire_gather(u1, 1, gsem1)
        transpose(0)
        fire_out(u0, 0, osem0)
        drain_gather(u1, 1, gsem1)

        @pl.when(up + 1 < _NU // 2)
        def _():
            fire_gather(u0 + 2, 0, gsem0)

        transpose(1)
        fire_out(u1, 1, osem1)
        drain_out(u0, 0, osem0)
        drain_out(u1, 1, osem1)


def kernel(ids, length, table):
    del length
    out5 = _gather(ids.T, table)
    return jnp.transpose(out5, (2, 4, 0, 1, 3)).reshape(_BATCH, _HIST, _D)


# speedup vs baseline: 1.4134x; 1.0210x over previous
import functools

import jax
import jax.numpy as jnp
from jax import lax
from jax.experimental import pallas as pl
from jax.experimental.pallas import tpu as pltpu
from jax.experimental.pallas import tpu_sc as plsc

_VOCAB = 1000000
_D = 32
_BATCH = 16384
_HIST = 50
_NC = 2
_NS = 16
_NW = _NC * _NS
_BBLK = _BATCH // 128
_CPW = _BBLK // _NW
_NU = _HIST * _CPW

_mesh = plsc.VectorSubcoreMesh(core_axis_name="c", subcore_axis_name="s")


@functools.partial(
    pl.kernel,
    out_type=jax.ShapeDtypeStruct((_HIST, _D // 8, 128, 8, 128), jnp.float32),
    mesh=_mesh,
    scratch_types=[
        pltpu.VMEM((_HIST, _CPW * 128), jnp.int32),
        pltpu.VMEM((4, 128, _D), jnp.float32),
        pltpu.VMEM((4, _D // 8, 8, 128), jnp.float32),
        [pltpu.SemaphoreType.DMA for _ in range(4)],
        [pltpu.SemaphoreType.DMA for _ in range(4)],
    ],
    compiler_params=pltpu.CompilerParams(use_tc_tiling_on_sc=False, needs_layout_passes=False),
)
def _gather(ids_hbm, table_hbm, out_hbm, idx_v, rows_v, tblk_v,
            gsems, osems):
    wid = lax.axis_index("s") * _NC + lax.axis_index("c")
    col0 = wid * (_CPW * 128)
    pltpu.sync_copy(ids_hbm.at[:, pl.ds(col0, _CPW * 128)], idx_v)

    lanes = lax.iota(jnp.int32, 16)
    rvecs = [lanes + (bg * 16) for bg in range(8)]

    def fire_gather(u, half, sem):
        t = u // _CPW
        c = u % _CPW
        pltpu.async_copy(table_hbm.at[idx_v.at[t, pl.ds(c * 128, 128)]],
                         rows_v.at[half], sem)

    def drain_gather(u, half, sem):
        t = u // _CPW
        c = u % _CPW
        pltpu.make_async_copy(table_hbm.at[idx_v.at[t, pl.ds(c * 128, 128)]],
                              rows_v.at[half], sem).wait()

    def transpose(half):
        @pl.loop(0, _D // 16)
        def _j(jq):
            j0 = 16 * jq
            cvecs = [jnp.zeros((16,), jnp.int32) + (j0 + q) for q in range(16)]
            for bg in range(8):
                vals = [plsc.load_gather(rows_v.at[half], [rvecs[bg], cvecs[q]])
                        for q in range(16)]
                for q in range(16):
                    tblk_v.at[half, 2 * jq + q // 8, q % 8][pl.ds(bg * 16, 16)] = vals[q]

    def fire_out(u, half, sem):
        t = u // _CPW
        c = u % _CPW
        pltpu.async_copy(tblk_v.at[half],
                         out_hbm.at[t, :, wid * _CPW + c], sem)

    def drain_out(u, half, sem):
        t = u // _CPW
        c = u % _CPW
        pltpu.make_async_copy(tblk_v.at[half],
                              out_hbm.at[t, :, wid * _CPW + c], sem).wait()

    for q in range(4):
        fire_gather(q, q, gsems[q])

    @pl.loop(0, _NU // 4)
    def _quad(g):
        for q in range(4):
            u = 4 * g + q
            drain_gather(u, q, gsems[q])

            @pl.when(g > 0)
            def _():
                drain_out(u - 4, q, osems[q])

            transpose(q)

            @pl.when(g + 1 < _NU // 4)
            def _():
                fire_gather(u + 4, q, gsems[q])

            fire_out(u, q, osems[q])

    for q in range(4):
        drain_out(_NU - 4 + q, q, osems[q])


def kernel(ids, length, table):
    del length
    out5 = _gather(ids.T, table)
    return jnp.transpose(out5, (2, 4, 0, 1, 3)).reshape(_BATCH, _HIST, _D)
